# baseline (speedup 1.0000x reference)
import functools

import jax
import jax.numpy as jnp
from jax import lax
from jax.experimental import pallas as pl
from jax.experimental.pallas import tpu as pltpu
from jax.experimental.pallas import tpu_sc as plsc

REG_W = 0.01
NODES = 1000000
D = 64
B = 524288

NC = 2
NS = 16
L = 16
NW = NC * NS
CHUNK = 128
TILE_EDGES = B // NW
NCHUNK = TILE_EDGES // CHUNK
ROWS2D = B // CHUNK

EMB_BLOCK = 8000


def _sc_scores_body(table, heads, tails, w, out,
                    heads_v, tails_v, w_v, srow, orow, scores_v, q, sem):
    wid = lax.axis_index("s") * NC + lax.axis_index("c")
    base = wid * NCHUNK

    pltpu.sync_copy(heads.at[pl.ds(base, NCHUNK), :], heads_v)
    pltpu.sync_copy(tails.at[pl.ds(base, NCHUNK), :], tails_v)
    pltpu.sync_copy(w, w_v)
    r = [w_v[pl.ds(k * L, L)] for k in range(4)]

    row_iota = lax.iota(jnp.int32, L)

    def chunk_body(c, carry):
        cp1 = pltpu.async_copy(table.at[heads_v.at[c]], srow, sem)
        cp2 = pltpu.async_copy(table.at[tails_v.at[c]], orow, sem)
        cp1.wait()
        cp2.wait()
        for g in range(CHUNK // L):
            for j in range(L):
                e = g * L + j
                acc = (srow[e, pl.ds(0, L)] * r[0]) * orow[e, pl.ds(0, L)]
                for k in range(1, 4):
                    acc += (srow[e, pl.ds(k * L, L)] * r[k]) * orow[e, pl.ds(k * L, L)]
                q[j, :] = acc
            s = plsc.load_gather(q, [row_iota, jnp.zeros((L,), jnp.int32)])
            for l in range(1, L):
                s += plsc.load_gather(q, [row_iota, jnp.full((L,), l, jnp.int32)])
            scores_v[c, pl.ds(g * L, L)] = s
        return carry

    lax.fori_loop(0, NCHUNK, chunk_body, 0)
    pltpu.sync_copy(scores_v, out.at[pl.ds(base, NCHUNK), :])


@jax.jit
def _sc_scores(table, heads, tails, w):
    mesh = plsc.VectorSubcoreMesh(core_axis_name="c", subcore_axis_name="s")
    return pl.kernel(
        _sc_scores_body,
        out_type=jax.ShapeDtypeStruct((ROWS2D, CHUNK), jnp.float32),
        mesh=mesh,
        scratch_types=[
            pltpu.VMEM((NCHUNK, CHUNK), jnp.int32),
            pltpu.VMEM((NCHUNK, CHUNK), jnp.int32),
            pltpu.VMEM((D,), jnp.float32),
            pltpu.VMEM((CHUNK, D), jnp.float32),
            pltpu.VMEM((CHUNK, D), jnp.float32),
            pltpu.VMEM((NCHUNK, CHUNK), jnp.float32),
            pltpu.VMEM((L, L), jnp.float32),
            pltpu.SemaphoreType.DMA,
        ],
        compiler_params=pltpu.CompilerParams(
            needs_layout_passes=False, use_tc_tiling_on_sc=False),
    )(table, heads, tails, w)


def _tc_finish_body(emb_ref, scores_ref, labels_ref, w_ref, out_ref):
    i = pl.program_id(0)
    eblk = emb_ref[...]
    part = jnp.sum(eblk * eblk)

    @pl.when(i == 0)
    def _():
        s = scores_ref[...]
        y = labels_ref[...]
        bce = jnp.sum(jnp.maximum(s, 0.0) - s * y + jnp.log1p(jnp.exp(-jnp.abs(s))))
        wv = w_ref[...]
        base = bce / B + REG_W * (jnp.sum(wv * wv) / D)
        out_ref[...] = jnp.full((8, 128), base, jnp.float32)

    out_ref[...] += jnp.full((8, 128), REG_W * (part / (NODES * D)), jnp.float32)


@jax.jit
def _tc_finish(emb, scores, labels, w2d):
    return pl.pallas_call(
        _tc_finish_body,
        grid=(NODES // EMB_BLOCK,),
        in_specs=[
            pl.BlockSpec((EMB_BLOCK, D), lambda i: (i, 0)),
            pl.BlockSpec((ROWS2D, CHUNK), lambda i: (0, 0)),
            pl.BlockSpec((ROWS2D, CHUNK), lambda i: (0, 0)),
            pl.BlockSpec((1, D), lambda i: (0, 0)),
        ],
        out_specs=pl.BlockSpec((8, 128), lambda i: (0, 0)),
        out_shape=jax.ShapeDtypeStruct((8, 128), jnp.float32),
    )(emb, scores, labels, w2d)


def kernel(embed_node, pairs_rel0, labels_rel0, w_rel0):
    heads = pairs_rel0[:, 0].reshape(ROWS2D, CHUNK)
    tails = pairs_rel0[:, 1].reshape(ROWS2D, CHUNK)
    scores = _sc_scores(embed_node, heads, tails, w_rel0)
    labels2d = labels_rel0.reshape(ROWS2D, CHUNK)
    w2d = w_rel0.reshape(1, D)
    out = _tc_finish(embed_node, scores, labels2d, w2d)
    return out[0, 0]


# speedup vs baseline: 1.1587x; 1.1587x over previous
import functools

import jax
import jax.numpy as jnp
from jax import lax
from jax.experimental import pallas as pl
from jax.experimental.pallas import tpu as pltpu
from jax.experimental.pallas import tpu_sc as plsc

REG_W = 0.01
NODES = 1000000
D = 64
B = 524288

NC = 2
NS = 16
L = 16
NW = NC * NS
CHUNK = 128
TILE_EDGES = B // NW
NCHUNK = TILE_EDGES // CHUNK
ROWS2D = B // CHUNK

EMB_BLOCK = 25000


def _sc_scores_body(table, heads, tails, w, out,
                    heads_v, tails_v, w_v, srow, orow, scores_v, q, sem):
    wid = lax.axis_index("s") * NC + lax.axis_index("c")
    base = wid * NCHUNK

    pltpu.sync_copy(heads.at[pl.ds(base, NCHUNK), :], heads_v)
    pltpu.sync_copy(tails.at[pl.ds(base, NCHUNK), :], tails_v)
    pltpu.sync_copy(w, w_v)
    r = [w_v[pl.ds(k * L, L)] for k in range(4)]

    row_iota = lax.iota(jnp.int32, L)

    def chunk_body(c, carry):
        cp1 = pltpu.async_copy(table.at[heads_v.at[c]], srow, sem)
        cp2 = pltpu.async_copy(table.at[tails_v.at[c]], orow, sem)
        cp1.wait()
        cp2.wait()
        for g in range(CHUNK // L):
            for j in range(L):
                e = g * L + j
                acc = (srow[e, pl.ds(0, L)] * r[0]) * orow[e, pl.ds(0, L)]
                for k in range(1, 4):
                    acc += (srow[e, pl.ds(k * L, L)] * r[k]) * orow[e, pl.ds(k * L, L)]
                q[j, :] = acc
            s = plsc.load_gather(q, [row_iota, jnp.zeros((L,), jnp.int32)])
            for l in range(1, L):
                s += plsc.load_gather(q, [row_iota, jnp.full((L,), l, jnp.int32)])
            scores_v[c, pl.ds(g * L, L)] = s
        return carry

    lax.fori_loop(0, NCHUNK, chunk_body, 0)
    pltpu.sync_copy(scores_v, out.at[pl.ds(base, NCHUNK), :])


@jax.jit
def _sc_scores(table, heads, tails, w):
    mesh = plsc.VectorSubcoreMesh(core_axis_name="c", subcore_axis_name="s")
    return pl.kernel(
        _sc_scores_body,
        out_type=jax.ShapeDtypeStruct((ROWS2D, CHUNK), jnp.float32),
        mesh=mesh,
        scratch_types=[
            pltpu.VMEM((NCHUNK, CHUNK), jnp.int32),
            pltpu.VMEM((NCHUNK, CHUNK), jnp.int32),
            pltpu.VMEM((D,), jnp.float32),
            pltpu.VMEM((CHUNK, D), jnp.float32),
            pltpu.VMEM((CHUNK, D), jnp.float32),
            pltpu.VMEM((NCHUNK, CHUNK), jnp.float32),
            pltpu.VMEM((L, L), jnp.float32),
            pltpu.SemaphoreType.DMA,
        ],
        compiler_params=pltpu.CompilerParams(
            needs_layout_passes=False, use_tc_tiling_on_sc=False),
    )(table, heads, tails, w)


def _tc_reduce_body(emb_ref, out_ref):
    i = pl.program_id(0)
    eblk = emb_ref[...]
    part = jnp.full((8, 128), jnp.sum(eblk * eblk), jnp.float32)

    @pl.when(i == 0)
    def _():
        out_ref[...] = jnp.zeros((8, 128), jnp.float32)

    out_ref[...] += part


@jax.jit
def _tc_reduce(emb):
    return pl.pallas_call(
        _tc_reduce_body,
        grid=(NODES // EMB_BLOCK,),
        in_specs=[pl.BlockSpec((EMB_BLOCK, D), lambda i: (i, 0))],
        out_specs=pl.BlockSpec((8, 128), lambda i: (0, 0)),
        out_shape=jax.ShapeDtypeStruct((8, 128), jnp.float32),
    )(emb)


def _tc_bce_body(scores_ref, labels_ref, w_ref, red_ref, out_ref):
    s = scores_ref[...]
    y = labels_ref[...]
    bce = jnp.sum(jnp.maximum(s, 0.0) - s * y + jnp.log1p(jnp.exp(-jnp.abs(s))))
    wv = w_ref[...]
    total = (bce / B + REG_W * (jnp.sum(wv * wv) / D)
             + REG_W * (red_ref[0, 0] / (NODES * D)))
    out_ref[...] = jnp.full((8, 128), total, jnp.float32)


@jax.jit
def _tc_bce(scores, labels, w2d, red):
    return pl.pallas_call(
        _tc_bce_body,
        out_shape=jax.ShapeDtypeStruct((8, 128), jnp.float32),
    )(scores, labels, w2d, red)


def kernel(embed_node, pairs_rel0, labels_rel0, w_rel0):
    heads = pairs_rel0[:, 0].reshape(ROWS2D, CHUNK)
    tails = pairs_rel0[:, 1].reshape(ROWS2D, CHUNK)
    red = _tc_reduce(embed_node)
    scores = _sc_scores(embed_node, heads, tails, w_rel0)
    labels2d = labels_rel0.reshape(ROWS2D, CHUNK)
    w2d = w_rel0.reshape(1, D)
    out = _tc_bce(scores, labels2d, w2d, red)
    return out[0, 0]
